# baseline (speedup 1.0000x reference)
import functools

import jax
import jax.numpy as jnp
from jax import lax
from jax.experimental import pallas as pl
from jax.experimental.pallas import tpu as pltpu
from jax.experimental.pallas import tpu_sc as plsc

VOCAB = 1001
VOCAB_PAD = 1024
EMB = 32
LAYERS0 = 256
LAYERS1 = 128
D_OUT = 64
D_PAD = 128
BATCH = 16384

NUM_CORES = 2
NUM_SUBCORES = 16
NW = NUM_CORES * NUM_SUBCORES
B_PER_W = BATCH // NW
CHUNK = 128
NCHUNK = B_PER_W // CHUNK
LANES = 16


def _mlp_body(tabt_ref, w1t_ref, b1_ref, w2t_ref, b2_ref, w3t_ref, b3_ref, out_ref):
    h = jnp.dot(w1t_ref[...], tabt_ref[...], preferred_element_type=jnp.float32)
    h = jnp.maximum(h + b1_ref[...][:, None], 0.0)
    h = jnp.dot(w2t_ref[...], h, preferred_element_type=jnp.float32)
    h = jnp.maximum(h + b2_ref[...][:, None], 0.0)
    h = jnp.dot(w3t_ref[...], h, preferred_element_type=jnp.float32)
    h = h + b3_ref[...][:, None]
    out_ref[:, pl.ds(0, D_OUT)] = h.T


def _mlp_table(tab, W1, b1, W2, b2, W3, b3):
    return pl.pallas_call(
        _mlp_body,
        grid=(1,),
        in_specs=[
            pl.BlockSpec((EMB, VOCAB_PAD), lambda i: (0, 0)),
            pl.BlockSpec((LAYERS0, EMB), lambda i: (0, 0)),
            pl.BlockSpec((LAYERS0,), lambda i: (0,)),
            pl.BlockSpec((LAYERS1, LAYERS0), lambda i: (0, 0)),
            pl.BlockSpec((LAYERS1,), lambda i: (0,)),
            pl.BlockSpec((D_OUT, LAYERS1), lambda i: (0, 0)),
            pl.BlockSpec((D_OUT,), lambda i: (0,)),
        ],
        out_specs=pl.BlockSpec((VOCAB_PAD, D_PAD), lambda i: (0, 0)),
        out_shape=jax.ShapeDtypeStruct((VOCAB_PAD, D_PAD), jnp.float32),
    )(tab.T, W1.T, b1, W2.T, b2, W3.T, b3)


@functools.cache
def _make_sc_gather():
    mesh = plsc.VectorSubcoreMesh(
        core_axis_name="c",
        subcore_axis_name="s",
        num_cores=NUM_CORES,
        num_subcores=NUM_SUBCORES,
    )

    @functools.partial(
        pl.kernel,
        mesh=mesh,
        compiler_params=pltpu.CompilerParams(use_tc_tiling_on_sc=False),
        out_type=jax.ShapeDtypeStruct((BATCH, D_PAD), jnp.float32),
        scratch_types=[
            pltpu.VMEM((NCHUNK, CHUNK), jnp.int32),
            pltpu.VMEM((B_PER_W, D_OUT), jnp.float32),
            pltpu.SemaphoreType.DMA,
        ],
    )
    def _sc_gather(tab_hbm, idx_hbm, out_hbm, idx_v, rows_v, sem):
        wid = lax.axis_index("s") * NUM_CORES + lax.axis_index("c")
        base = wid * B_PER_W
        for j in range(NCHUNK):
            pltpu.sync_copy(idx_hbm.at[pl.ds(base + j * CHUNK, CHUNK)], idx_v.at[j])
        for j in range(NCHUNK):
            for k in range(CHUNK // LANES):
                sl = pl.ds(k * LANES, LANES)
                idx_v[j, sl] = idx_v[j, sl] * 2
        copies = [
            pltpu.async_copy(
                tab_hbm.at[idx_v.at[j]],
                rows_v.at[pl.ds(j * CHUNK, CHUNK)],
                sem,
            )
            for j in range(NCHUNK)
        ]
        for c in copies:
            c.wait()
        pltpu.sync_copy(rows_v, out_hbm.at[pl.ds(base, B_PER_W), pl.ds(0, D_OUT)])

    return _sc_gather


def kernel(indices, table, W1, b1, W2, b2, W3, b3):
    idx = indices.astype(jnp.int32)
    out_table = _mlp_table(table, W1, b1, W2, b2, W3, b3)
    tab_view = out_table.reshape(2 * VOCAB_PAD, D_OUT)
    gathered = _make_sc_gather()(tab_view, idx)
    return gathered[:, :D_OUT]


# speedup vs baseline: 1.1039x; 1.1039x over previous
import functools

import jax
import jax.numpy as jnp
from jax import lax
from jax.experimental import pallas as pl
from jax.experimental.pallas import tpu as pltpu
from jax.experimental.pallas import tpu_sc as plsc

VOCAB = 1001
VOCAB_PAD = 1024
EMB = 32
LAYERS0 = 256
LAYERS1 = 128
D_OUT = 64
D_PAD = 128
BATCH = 16384

NUM_CORES = 2
NUM_SUBCORES = 16
NW = NUM_CORES * NUM_SUBCORES
B_PER_W = BATCH // NW
CHUNK = 128
NCHUNK = B_PER_W // CHUNK
LANES = 16


def _mlp_body(tabt_ref, w1_ref, b1_ref, w2_ref, b2_ref, w3t_ref, b3_ref, out_ref):
    dn0 = (((0,), (0,)), ((), ()))
    h = jax.lax.dot_general(
        w1_ref[...], tabt_ref[...], dn0, preferred_element_type=jnp.float32
    )
    h = jnp.maximum(h + b1_ref[...][:, None], 0.0)
    h = jax.lax.dot_general(
        w2_ref[...], h, dn0, preferred_element_type=jnp.float32
    )
    h = jnp.maximum(h + b2_ref[...][:, None], 0.0)
    h = jnp.dot(w3t_ref[...], h, preferred_element_type=jnp.float32)
    h = h + b3_ref[...][:, None]
    out_ref[:, pl.ds(0, D_OUT)] = h.T


def _mlp_table(tab, W1, b1, W2, b2, W3, b3):
    return pl.pallas_call(
        _mlp_body,
        grid=(1,),
        in_specs=[
            pl.BlockSpec((EMB, VOCAB_PAD), lambda i: (0, 0)),
            pl.BlockSpec((EMB, LAYERS0), lambda i: (0, 0)),
            pl.BlockSpec((LAYERS0,), lambda i: (0,)),
            pl.BlockSpec((LAYERS0, LAYERS1), lambda i: (0, 0)),
            pl.BlockSpec((LAYERS1,), lambda i: (0,)),
            pl.BlockSpec((D_OUT, LAYERS1), lambda i: (0, 0)),
            pl.BlockSpec((D_OUT,), lambda i: (0,)),
        ],
        out_specs=pl.BlockSpec((VOCAB_PAD, D_PAD), lambda i: (0, 0)),
        out_shape=jax.ShapeDtypeStruct((VOCAB_PAD, D_PAD), jnp.float32),
    )(tab.T, W1, b1, W2, b2, W3.T, b3)


@functools.cache
def _make_sc_gather():
    mesh = plsc.VectorSubcoreMesh(
        core_axis_name="c",
        subcore_axis_name="s",
        num_cores=NUM_CORES,
        num_subcores=NUM_SUBCORES,
    )

    @functools.partial(
        pl.kernel,
        mesh=mesh,
        compiler_params=pltpu.CompilerParams(use_tc_tiling_on_sc=False),
        out_type=jax.ShapeDtypeStruct((BATCH, D_PAD), jnp.float32),
        scratch_types=[
            pltpu.VMEM((NCHUNK, CHUNK), jnp.int32),
            pltpu.VMEM((B_PER_W, D_OUT), jnp.float32),
            pltpu.SemaphoreType.DMA,
        ],
    )
    def _sc_gather(tab_hbm, idx_hbm, out_hbm, idx_v, rows_v, sem):
        wid = lax.axis_index("s") * NUM_CORES + lax.axis_index("c")
        base = wid * B_PER_W
        for j in range(NCHUNK):
            pltpu.sync_copy(idx_hbm.at[pl.ds(base + j * CHUNK, CHUNK)], idx_v.at[j])
        for j in range(NCHUNK):
            for k in range(CHUNK // LANES):
                sl = pl.ds(k * LANES, LANES)
                idx_v[j, sl] = idx_v[j, sl] * 2
        copies = [
            pltpu.async_copy(
                tab_hbm.at[idx_v.at[j]],
                rows_v.at[pl.ds(j * CHUNK, CHUNK)],
                sem,
            )
            for j in range(NCHUNK)
        ]
        for c in copies:
            c.wait()
        pltpu.sync_copy(rows_v, out_hbm.at[pl.ds(base, B_PER_W), pl.ds(0, D_OUT)])

    return _sc_gather


def kernel(indices, table, W1, b1, W2, b2, W3, b3):
    idx = indices.astype(jnp.int32)
    out_table = _mlp_table(table, W1, b1, W2, b2, W3, b3)
    tab_view = out_table.reshape(2 * VOCAB_PAD, D_OUT)
    gathered = _make_sc_gather()(tab_view, idx)
    return gathered[:, :D_OUT]
